# baseline (speedup 1.0000x reference)
import functools

import jax
import jax.numpy as jnp
import numpy as np
from jax import lax
from jax.experimental import pallas as pl
from jax.experimental.pallas import tpu as pltpu
from jax.experimental.pallas import tpu_sc as plsc

NC = 2
NS = 16
NW = NC * NS
CH = 128
LAN = 16


def _wid():
    return lax.axis_index("s") * NC + lax.axis_index("c")


def _make_dual_gather(EP, CPW, D1, D2):
    mesh = plsc.VectorSubcoreMesh(core_axis_name="c", subcore_axis_name="s")

    @functools.partial(
        pl.kernel,
        out_type=(jax.ShapeDtypeStruct((EP, D1), jnp.float32),
                  jax.ShapeDtypeStruct((EP, D2), jnp.float32)),
        mesh=mesh,
        scratch_types=[
            pltpu.VMEM((CPW, CH), jnp.int32),
            pltpu.VMEM((CPW, CH), jnp.int32),
            pltpu.VMEM((CH, D1), jnp.float32),
            pltpu.VMEM((CH, D2), jnp.float32),
            pltpu.SemaphoreType.DMA,
            pltpu.SemaphoreType.DMA,
        ],
    )
    def k(t1, t2, idx1, idx2, o1, o2, idx1_v, idx2_v, b1, b2, s1, s2):
        w = _wid()
        pltpu.sync_copy(idx1.at[w], idx1_v)
        pltpu.sync_copy(idx2.at[w], idx2_v)

        def chunk(j, carry):
            base = (w * CPW + j) * CH
            c1 = pltpu.async_copy(t1.at[idx1_v.at[j]], b1, s1)
            c2 = pltpu.async_copy(t2.at[idx2_v.at[j]], b2, s2)
            c1.wait()
            c2.wait()
            pltpu.sync_copy(b1, o1.at[pl.ds(base, CH)])
            pltpu.sync_copy(b2, o2.at[pl.ds(base, CH)])
            return carry

        lax.fori_loop(0, CPW, chunk, 0)

    return k


def _make_scatter_add(EP, CPW, D, NPAD):
    RPT = NPAD // NS
    assert RPT % CH == 0
    mesh = plsc.VectorSubcoreMesh(core_axis_name="c", subcore_axis_name="s")

    @functools.partial(
        pl.kernel,
        out_type=jax.ShapeDtypeStruct((NC, NPAD, D), jnp.float32),
        mesh=mesh,
        scratch_types=[
            pltpu.VMEM((CPW, CH), jnp.int32),
            pltpu.VMEM((CH, D), jnp.float32),
            pltpu.VMEM((CH, D), jnp.float32),
            pltpu.VMEM_SHARED((NPAD, D), jnp.float32),
            pltpu.SemaphoreType.DMA,
        ],
    )
    def k(m, cidx, out, cidx_v, mbuf, zbuf, acc, lsem):
        c = lax.axis_index("c")
        s = lax.axis_index("s")
        w = s * NC + c
        zero = jnp.zeros((LAN,), jnp.float32)

        def zrow(r, carry):
            for kk in range(D // LAN):
                zbuf[r, pl.ds(kk * LAN, LAN)] = zero
            return carry

        lax.fori_loop(0, CH, zrow, 0)
        for i in range(RPT // CH):
            pltpu.sync_copy(zbuf, acc.at[pl.ds(s * RPT + i * CH, CH)])
        plsc.subcore_barrier()

        pltpu.sync_copy(cidx.at[w], cidx_v)

        def chunk(j, carry):
            base = (w * CPW + j) * CH
            pltpu.sync_copy(m.at[pl.ds(base, CH)], mbuf)
            pltpu.sync_copy(mbuf, acc.at[cidx_v.at[j]], add=True)
            return carry

        lax.fori_loop(0, CPW, chunk, 0)
        plsc.subcore_barrier()
        for i in range(RPT // CH):
            r0 = s * RPT + i * CH
            pltpu.sync_copy(acc.at[pl.ds(r0, CH)], out.at[c].at[pl.ds(r0, CH)])

    return k


_DOT = functools.partial(jnp.dot, preferred_element_type=jnp.float32,
                         precision=lax.Precision.HIGHEST)


def _ln(x, g, b):
    mu = jnp.mean(x, axis=1, keepdims=True)
    xc = x - mu
    var = jnp.mean(xc * xc, axis=1, keepdims=True)
    return xc * lax.rsqrt(var + 1e-5) * g + b


def _tc_ab(h, wa, wb, bm=400):
    M, D = h.shape

    def body(h_ref, wa_ref, wb_ref, a_ref, b_ref):
        hh = h_ref[...]
        a_ref[...] = _DOT(hh, wa_ref[...])
        b_ref[...] = _DOT(hh, wb_ref[...])

    return pl.pallas_call(
        body,
        grid=(M // bm,),
        in_specs=[pl.BlockSpec((bm, D), lambda i: (i, 0)),
                  pl.BlockSpec((D, D), lambda i: (0, 0)),
                  pl.BlockSpec((D, D), lambda i: (0, 0))],
        out_specs=[pl.BlockSpec((bm, D), lambda i: (i, 0))] * 2,
        out_shape=[jax.ShapeDtypeStruct((M, D), jnp.float32)] * 2,
    )(h, wa, wb)


def _tc_dist(gpr, gpc, be=512):
    EP, DP = gpr.shape

    def body(pr_ref, pc_ref, o_ref):
        df = pr_ref[...] - pc_ref[...]
        o_ref[...] = jnp.sqrt(jnp.sum(df * df, axis=1, keepdims=True))

    return pl.pallas_call(
        body,
        grid=(EP // be,),
        in_specs=[pl.BlockSpec((be, DP), lambda i: (i, 0))] * 2,
        out_specs=pl.BlockSpec((be, 1), lambda i: (i, 0)),
        out_shape=jax.ShapeDtypeStruct((EP, 1), jnp.float32),
    )(gpr, gpc)


def _tc_msg(ga, gb, dist, we, b, g, bb, be=512):
    EP, D = ga.shape
    inv_sqrt2 = np.float32(1.0 / np.sqrt(2.0))

    def body(ga_ref, gb_ref, d_ref, we_ref, b_ref, g_ref, bb_ref, o_ref):
        pre = ga_ref[...] + gb_ref[...] + d_ref[...] * we_ref[...] + b_ref[...]
        m = 0.5 * pre * (1.0 + lax.erf(pre * inv_sqrt2))
        o_ref[...] = _ln(m, g_ref[...], bb_ref[...])

    vec = pl.BlockSpec((1, D), lambda i: (0, 0))
    return pl.pallas_call(
        body,
        grid=(EP // be,),
        in_specs=[pl.BlockSpec((be, D), lambda i: (i, 0)),
                  pl.BlockSpec((be, D), lambda i: (i, 0)),
                  pl.BlockSpec((be, 1), lambda i: (i, 0)),
                  vec, vec, vec, vec],
        out_specs=pl.BlockSpec((be, D), lambda i: (i, 0)),
        out_shape=jax.ShapeDtypeStruct((EP, D), jnp.float32),
    )(ga, gb, dist, we, b, g, bb)


def _tc_update(h, part, cntp, wu1, wu2, b, g, bb, bm=400):
    M, D = h.shape
    DC = cntp.shape[2]

    def body(h_ref, p_ref, c_ref, wu1_ref, wu2_ref, b_ref, g_ref, bb_ref, o_ref):
        hh = h_ref[...]
        p = p_ref[0] + p_ref[1]
        cnt = jnp.sum(c_ref[0] + c_ref[1], axis=1, keepdims=True) * (1.0 / DC)
        agg = p / jnp.maximum(cnt, 1.0)
        u = _DOT(hh, wu1_ref[...]) + _DOT(agg, wu2_ref[...]) + b_ref[...]
        o_ref[...] = hh + _ln(u, g_ref[...], bb_ref[...])

    vec = pl.BlockSpec((1, D), lambda i: (0, 0))
    return pl.pallas_call(
        body,
        grid=(M // bm,),
        in_specs=[pl.BlockSpec((bm, D), lambda i: (i, 0)),
                  pl.BlockSpec((2, bm, D), lambda i: (0, i, 0)),
                  pl.BlockSpec((2, bm, DC), lambda i: (0, i, 0)),
                  pl.BlockSpec((D, D), lambda i: (0, 0)),
                  pl.BlockSpec((D, D), lambda i: (0, 0)),
                  vec, vec, vec],
        out_specs=pl.BlockSpec((bm, D), lambda i: (i, 0)),
        out_shape=jax.ShapeDtypeStruct((M, D), jnp.float32),
    )(h, part, cntp, wu1, wu2, b, g, bb)


def kernel(x, pos, edge_index, w_msg, b_msg, g_msg, be_msg,
           w_upd, b_upd, g_upd, be_upd):
    N, D = x.shape
    E = edge_index.shape[1]
    S = w_msg.shape[0]

    CPW = -(-E // (NW * CH))
    CPW = -(-CPW // 3) * 3
    EP = NW * CPW * CH
    NPAD = -(-(N + 1) // (NS * CH)) * (NS * CH)

    row = edge_index[0]
    col = edge_index[1]
    pad = EP - E
    row_g = jnp.pad(row, (0, pad)).reshape(NW, CPW, CH)
    col_g = jnp.pad(col, (0, pad)).reshape(NW, CPW, CH)
    col_s = jnp.pad(col, (0, pad), constant_values=N).reshape(NW, CPW, CH)

    wa = w_msg[:, :D, :]
    wb = w_msg[:, D:2 * D, :]
    we = w_msg[:, 2 * D, :].reshape(S, 1, D)
    wu1 = w_upd[:, :D, :]
    wu2 = w_upd[:, D:, :]

    pos128 = jnp.pad(pos, ((0, 0), (0, D - pos.shape[1])))

    gather_ab = _make_dual_gather(EP, CPW, D, D)
    scatter_m = _make_scatter_add(EP, CPW, D, NPAD)

    gpr, gpc = gather_ab(pos128, pos128, row_g, col_g)
    dist = _tc_dist(gpr, gpc)
    cntp = scatter_m(jnp.ones((EP, D), jnp.float32), col_s)

    h = x
    for s in range(S):
        a, bt = _tc_ab(h, wa[s], wb[s])
        ga, gb = gather_ab(a, bt, row_g, col_g)
        m = _tc_msg(ga, gb, dist, we[s], b_msg[s][None], g_msg[s][None],
                    be_msg[s][None])
        part = scatter_m(m, col_s)
        h = _tc_update(h, part, cntp, wu1[s], wu2[s], b_upd[s][None],
                       g_upd[s][None], be_upd[s][None])
    return h


# speedup vs baseline: 1.5083x; 1.5083x over previous
import functools

import jax
import jax.numpy as jnp
import numpy as np
from jax import lax
from jax.experimental import pallas as pl
from jax.experimental.pallas import tpu as pltpu
from jax.experimental.pallas import tpu_sc as plsc

NC = 2
NS = 16
NW = NC * NS
CH = 128
LAN = 16


def _wid():
    return lax.axis_index("s") * NC + lax.axis_index("c")


def _make_dual_gather(EP, CPW, D1, D2):
    mesh = plsc.VectorSubcoreMesh(core_axis_name="c", subcore_axis_name="s")

    @functools.partial(
        pl.kernel,
        out_type=(jax.ShapeDtypeStruct((EP, D1), jnp.float32),
                  jax.ShapeDtypeStruct((EP, D2), jnp.float32)),
        mesh=mesh,
        scratch_types=[
            pltpu.VMEM((CPW, CH), jnp.int32),
            pltpu.VMEM((CPW, CH), jnp.int32),
            pltpu.VMEM((CH, D1), jnp.float32),
            pltpu.VMEM((CH, D2), jnp.float32),
            pltpu.SemaphoreType.DMA,
            pltpu.SemaphoreType.DMA,
        ],
    )
    def k(t1, t2, idx1, idx2, o1, o2, idx1_v, idx2_v, b1, b2, s1, s2):
        w = _wid()
        pltpu.sync_copy(idx1.at[w], idx1_v)
        pltpu.sync_copy(idx2.at[w], idx2_v)

        def chunk(j, carry):
            base = (w * CPW + j) * CH
            c1 = pltpu.async_copy(t1.at[idx1_v.at[j]], b1, s1)
            c2 = pltpu.async_copy(t2.at[idx2_v.at[j]], b2, s2)
            c1.wait()
            c2.wait()
            pltpu.sync_copy(b1, o1.at[pl.ds(base, CH)])
            pltpu.sync_copy(b2, o2.at[pl.ds(base, CH)])
            return carry

        lax.fori_loop(0, CPW, chunk, 0)

    return k


def _make_scatter_add(EP, CPW, D, NPAD):
    RPT = NPAD // NS
    assert RPT % CH == 0
    mesh = plsc.VectorSubcoreMesh(core_axis_name="c", subcore_axis_name="s")

    @functools.partial(
        pl.kernel,
        out_type=jax.ShapeDtypeStruct((NC, NPAD, D), jnp.float32),
        mesh=mesh,
        scratch_types=[
            pltpu.VMEM((CPW, CH), jnp.int32),
            pltpu.VMEM((CH, D), jnp.float32),
            pltpu.VMEM((CH, D), jnp.float32),
            pltpu.VMEM_SHARED((NPAD, D), jnp.float32),
            pltpu.SemaphoreType.DMA,
        ],
    )
    def k(m, cidx, out, cidx_v, mbuf, zbuf, acc, lsem):
        c = lax.axis_index("c")
        s = lax.axis_index("s")
        w = s * NC + c
        zero = jnp.zeros((LAN,), jnp.float32)

        def zrow(r, carry):
            for kk in range(D // LAN):
                zbuf[r, pl.ds(kk * LAN, LAN)] = zero
            return carry

        lax.fori_loop(0, CH, zrow, 0)
        for i in range(RPT // CH):
            pltpu.sync_copy(zbuf, acc.at[pl.ds(s * RPT + i * CH, CH)])
        plsc.subcore_barrier()

        pltpu.sync_copy(cidx.at[w], cidx_v)

        def chunk(j, carry):
            base = (w * CPW + j) * CH
            pltpu.sync_copy(m.at[pl.ds(base, CH)], mbuf)
            pltpu.sync_copy(mbuf, acc.at[cidx_v.at[j]], add=True)
            return carry

        lax.fori_loop(0, CPW, chunk, 0)
        plsc.subcore_barrier()
        for i in range(RPT // CH):
            r0 = s * RPT + i * CH
            pltpu.sync_copy(acc.at[pl.ds(r0, CH)], out.at[c].at[pl.ds(r0, CH)])

    return k


def _make_count(EP, CPW, NPAD, D):
    RPT = NPAD // NS
    mesh = plsc.VectorSubcoreMesh(core_axis_name="c", subcore_axis_name="s")

    @functools.partial(
        pl.kernel,
        out_type=jax.ShapeDtypeStruct((NC, NPAD, D), jnp.float32),
        mesh=mesh,
        scratch_types=[
            pltpu.VMEM((CPW, CH), jnp.int32),
            pltpu.VMEM((CH, D), jnp.float32),
            pltpu.VMEM((CH, D), jnp.float32),
            pltpu.VMEM_SHARED((NPAD, D), jnp.float32),
        ],
    )
    def k(cidx, out, cidx_v, onebuf, zbuf, acc):
        c = lax.axis_index("c")
        s = lax.axis_index("s")
        w = s * NC + c
        zero = jnp.zeros((LAN,), jnp.float32)
        one = jnp.ones((LAN,), jnp.float32)

        def frow(r, carry):
            for kk in range(D // LAN):
                zbuf[r, pl.ds(kk * LAN, LAN)] = zero
                onebuf[r, pl.ds(kk * LAN, LAN)] = one
            return carry

        lax.fori_loop(0, CH, frow, 0)
        for i in range(RPT // CH):
            pltpu.sync_copy(zbuf, acc.at[pl.ds(s * RPT + i * CH, CH)])
        plsc.subcore_barrier()

        pltpu.sync_copy(cidx.at[w], cidx_v)

        def chunk(j, carry):
            pltpu.sync_copy(onebuf, acc.at[cidx_v.at[j]], add=True)
            return carry

        lax.fori_loop(0, CPW, chunk, 0)
        plsc.subcore_barrier()
        for i in range(RPT // CH):
            r0 = s * RPT + i * CH
            pltpu.sync_copy(acc.at[pl.ds(r0, CH)], out.at[c].at[pl.ds(r0, CH)])

    return k


_DOT = functools.partial(jnp.dot, preferred_element_type=jnp.float32,
                         precision=lax.Precision.HIGHEST)


def _ln(x, g, b):
    mu = jnp.mean(x, axis=1, keepdims=True)
    xc = x - mu
    var = jnp.mean(xc * xc, axis=1, keepdims=True)
    return xc * lax.rsqrt(var + 1e-5) * g + b


def _tc_ab(h, wa, wb, bm=400):
    M, D = h.shape

    def body(h_ref, wa_ref, wb_ref, a_ref, b_ref):
        hh = h_ref[...]
        a_ref[...] = _DOT(hh, wa_ref[...])
        b_ref[...] = _DOT(hh, wb_ref[...])

    return pl.pallas_call(
        body,
        grid=(M // bm,),
        in_specs=[pl.BlockSpec((bm, D), lambda i: (i, 0)),
                  pl.BlockSpec((D, D), lambda i: (0, 0)),
                  pl.BlockSpec((D, D), lambda i: (0, 0))],
        out_specs=[pl.BlockSpec((bm, D), lambda i: (i, 0))] * 2,
        out_shape=[jax.ShapeDtypeStruct((M, D), jnp.float32)] * 2,
    )(h, wa, wb)


def _tc_dist(gpr, gpc, be=512):
    EP, DP = gpr.shape

    def body(pr_ref, pc_ref, o_ref):
        df = pr_ref[...] - pc_ref[...]
        o_ref[...] = jnp.sqrt(jnp.sum(df * df, axis=1, keepdims=True))

    return pl.pallas_call(
        body,
        grid=(EP // be,),
        in_specs=[pl.BlockSpec((be, DP), lambda i: (i, 0))] * 2,
        out_specs=pl.BlockSpec((be, 1), lambda i: (i, 0)),
        out_shape=jax.ShapeDtypeStruct((EP, 1), jnp.float32),
    )(gpr, gpc)


def _tc_msg(ga, gb, dist, we, b, g, bb, be=512):
    EP, D = ga.shape
    inv_sqrt2 = np.float32(1.0 / np.sqrt(2.0))

    def body(ga_ref, gb_ref, d_ref, we_ref, b_ref, g_ref, bb_ref, o_ref):
        pre = ga_ref[...] + gb_ref[...] + d_ref[...] * we_ref[...] + b_ref[...]
        m = 0.5 * pre * (1.0 + lax.erf(pre * inv_sqrt2))
        o_ref[...] = _ln(m, g_ref[...], bb_ref[...])

    vec = pl.BlockSpec((1, D), lambda i: (0, 0))
    return pl.pallas_call(
        body,
        grid=(EP // be,),
        in_specs=[pl.BlockSpec((be, D), lambda i: (i, 0)),
                  pl.BlockSpec((be, D), lambda i: (i, 0)),
                  pl.BlockSpec((be, 1), lambda i: (i, 0)),
                  vec, vec, vec, vec],
        out_specs=pl.BlockSpec((be, D), lambda i: (i, 0)),
        out_shape=jax.ShapeDtypeStruct((EP, D), jnp.float32),
    )(ga, gb, dist, we, b, g, bb)


def _tc_update(h, part, cntp, wu1, wu2, b, g, bb, bm=400):
    M, D = h.shape
    DC = cntp.shape[2]

    def body(h_ref, p_ref, c_ref, wu1_ref, wu2_ref, b_ref, g_ref, bb_ref, o_ref):
        hh = h_ref[...]
        p = p_ref[0] + p_ref[1]
        cnt = jnp.sum(c_ref[0] + c_ref[1], axis=1, keepdims=True) * (1.0 / DC)
        agg = p / jnp.maximum(cnt, 1.0)
        u = _DOT(hh, wu1_ref[...]) + _DOT(agg, wu2_ref[...]) + b_ref[...]
        o_ref[...] = hh + _ln(u, g_ref[...], bb_ref[...])

    vec = pl.BlockSpec((1, D), lambda i: (0, 0))
    return pl.pallas_call(
        body,
        grid=(M // bm,),
        in_specs=[pl.BlockSpec((bm, D), lambda i: (i, 0)),
                  pl.BlockSpec((2, bm, D), lambda i: (0, i, 0)),
                  pl.BlockSpec((2, bm, DC), lambda i: (0, i, 0)),
                  pl.BlockSpec((D, D), lambda i: (0, 0)),
                  pl.BlockSpec((D, D), lambda i: (0, 0)),
                  vec, vec, vec],
        out_specs=pl.BlockSpec((bm, D), lambda i: (i, 0)),
        out_shape=jax.ShapeDtypeStruct((M, D), jnp.float32),
    )(h, part, cntp, wu1, wu2, b, g, bb)


def kernel(x, pos, edge_index, w_msg, b_msg, g_msg, be_msg,
           w_upd, b_upd, g_upd, be_upd):
    N, D = x.shape
    E = edge_index.shape[1]
    S = w_msg.shape[0]

    CPW = -(-E // (NW * CH))
    EP = NW * CPW * CH
    NPAD = -(-(N + 1) // (NS * CH)) * (NS * CH)

    row = edge_index[0]
    col = edge_index[1]
    pad = EP - E
    row_g = jnp.pad(row, (0, pad)).reshape(NW, CPW, CH)
    col_g = jnp.pad(col, (0, pad)).reshape(NW, CPW, CH)
    col_s = jnp.pad(col, (0, pad), constant_values=N).reshape(NW, CPW, CH)

    wa = w_msg[:, :D, :]
    wb = w_msg[:, D:2 * D, :]
    we = w_msg[:, 2 * D, :].reshape(S, 1, D)
    wu1 = w_upd[:, :D, :]
    wu2 = w_upd[:, D:, :]

    pos128 = jnp.pad(pos, ((0, 0), (0, D - pos.shape[1])))

    gather_ab = _make_dual_gather(EP, CPW, D, D)
    scatter_m = _make_scatter_add(EP, CPW, D, NPAD)
    count_k = _make_count(EP, CPW, NPAD, D)

    gpr, gpc = gather_ab(pos128, pos128, row_g, col_g)
    dist = _tc_dist(gpr, gpc)
    cntp = count_k(col_s)

    h = x
    for s in range(S):
        a, bt = _tc_ab(h, wa[s], wb[s])
        ga, gb = gather_ab(a, bt, row_g, col_g)
        m = _tc_msg(ga, gb, dist, we[s], b_msg[s][None], g_msg[s][None],
                    be_msg[s][None])
        part = scatter_m(m, col_s)
        h = _tc_update(h, part, cntp, wu1[s], wu2[s], b_upd[s][None],
                       g_upd[s][None], be_upd[s][None])
    return h
